# Initial kernel scaffold; baseline (speedup 1.0000x reference)
import jax
import jax.numpy as jnp
from jax.experimental import pallas as pl


def kernel(x, edge_index, batch, W1, b1, W2, b2, W3, b3):
    raise NotImplementedError("write your pallas kernel here")


# speedup vs baseline: 13.3687x; 13.3687x over previous
import functools

import jax
import jax.numpy as jnp
from jax import lax
from jax.experimental import pallas as pl
from jax.experimental.pallas import tpu as pltpu
from jax.experimental.pallas import tpu_sc as plsc

N = 10000
E = 320000
NC = 2
NS = 16
NW = NC * NS
CH = 128
CHUNKS_W = 80
CG = 40
E_PAD = NW * CH * CHUNKS_W
DW = 16
EP_ROWS = E_PAD // CH
N_ACC = 10240
ROWS_T = N_ACC // NS
HOPS = ROWS_T // CH

@functools.cache
def _get_mesh():
    return plsc.VectorSubcoreMesh(core_axis_name="c", subcore_axis_name="s",
                                  num_cores=NC, num_subcores=NS)


def _zero_acc(z_hbm, stage, acc, sid):
    pltpu.sync_copy(z_hbm, stage)
    for j in range(HOPS):
        pltpu.sync_copy(stage, acc.at[pl.ds(sid * ROWS_T + j * CH, CH)])


def _drain_acc(acc, stage, out_hbm, cid, sid):
    for j in range(HOPS):
        sl = pl.ds(sid * ROWS_T + j * CH, CH)
        pltpu.sync_copy(acc.at[sl], stage)
        pltpu.sync_copy(stage, out_hbm.at[cid, sl])


@functools.cache
def _make_prop_kernel(D):
    @functools.partial(
        pl.kernel,
        out_type=jax.ShapeDtypeStruct((NC, N_ACC, D), jnp.float32),
        mesh=_get_mesh(),
        scratch_types=[
            pltpu.VMEM((CG, CH), jnp.int32),
            pltpu.VMEM((CG, CH), jnp.int32),
            pltpu.VMEM((CH, D), jnp.float32),
            pltpu.VMEM((CH, D), jnp.float32),
            pltpu.VMEM_SHARED((N_ACC, D), jnp.float32),
            pltpu.SemaphoreType.DMA,
            pltpu.SemaphoreType.DMA,
        ],
    )
    def prop_kernel(g_hbm, rowg_hbm, cols_hbm, z_hbm, out_hbm,
                    idxr, idxc, buf0, buf1, acc, sem0, sem1):
        cid = lax.axis_index("c")
        sid = lax.axis_index("s")
        wid = sid * NC + cid
        base = wid * CHUNKS_W
        _zero_acc(z_hbm, buf0, acc, sid)
        plsc.subcore_barrier()

        def step(c, buf, sem, start_next):
            pltpu.make_async_copy(g_hbm.at[idxr.at[c]], buf, sem).wait()
            pltpu.sync_copy(buf, acc.at[idxc.at[c]], add=True)
            if start_next:
                pltpu.async_copy(g_hbm.at[idxr.at[c + 2]], buf, sem)

        def body(k, _):
            c = 2 * k
            step(c, buf0, sem0, True)
            step(c + 1, buf1, sem1, True)
            return 0

        for gi in range(CHUNKS_W // CG):
            pltpu.sync_copy(rowg_hbm.at[pl.ds(base + gi * CG, CG)], idxr)
            pltpu.sync_copy(cols_hbm.at[pl.ds(base + gi * CG, CG)], idxc)
            pltpu.async_copy(g_hbm.at[idxr.at[0]], buf0, sem0)
            pltpu.async_copy(g_hbm.at[idxr.at[1]], buf1, sem1)
            lax.fori_loop(0, CG // 2 - 1, body, 0)
            step(CG - 2, buf0, sem0, False)
            step(CG - 1, buf1, sem1, False)
        plsc.subcore_barrier()
        _drain_acc(acc, buf0, out_hbm, cid, sid)

    return prop_kernel


_RB = 256
_R = 400
_RD = 640


def _prep_body(row_ref, col_ref, rowg_ref, rowd_ref, cols_ref):
    i = pl.program_id(0)
    r = row_ref[...]
    c = col_ref[...]
    ri = lax.broadcasted_iota(jnp.int32, (_RB, CH), 0)
    ci = lax.broadcasted_iota(jnp.int32, (_RB, CH), 1)
    pos = (i * _RB + ri) * CH + ci
    self_m = r == c
    trash = N + (pos & 127)
    spread = pos & 8191
    rowg_ref[...] = jnp.where(self_m, spread, r)
    rowd_ref[...] = jnp.where(self_m, trash, r)
    cols_ref[...] = jnp.where(self_m, trash, c)


_prep_call = pl.pallas_call(
    _prep_body,
    grid=(EP_ROWS // _RB,),
    in_specs=[pl.BlockSpec((_RB, CH), lambda i: (i, 0))] * 2,
    out_specs=[pl.BlockSpec((_RB, CH), lambda i: (i, 0))] * 3,
    out_shape=[jax.ShapeDtypeStruct((EP_ROWS, CH), jnp.int32)] * 3,
)


def _dis_g0_body(degp_ref, x_ref, dis_ref, g_ref):
    deg = (degp_ref[0] + degp_ref[1])[:, 0:1]
    ok = deg > 0.0
    dis = jnp.where(ok, lax.rsqrt(jnp.where(ok, deg, 1.0)), 0.0)
    dis_ref[...] = jnp.broadcast_to(dis, (dis.shape[0], DW))
    g_ref[...] = x_ref[...] * dis


def _make_dis_g0(Din):
    return pl.pallas_call(
        _dis_g0_body,
        grid=(N_ACC // _RD,),
        in_specs=[
            pl.BlockSpec((2, _RD, DP), lambda i: (0, i, 0)),
            pl.BlockSpec((_RD, Din), lambda i: (i, 0)),
        ],
        out_specs=[
            pl.BlockSpec((_RD, DW), lambda i: (i, 0)),
            pl.BlockSpec((_RD, Din), lambda i: (i, 0)),
        ],
        out_shape=[
            jax.ShapeDtypeStruct((N_ACC, DW), jnp.float32),
            jax.ShapeDtypeStruct((N, Din), jnp.float32),
        ],
    )


DP = 128


def _padw(v):
    if v.shape[1] == DP:
        return v
    return jnp.concatenate(
        [v, jnp.zeros((v.shape[0], DP - v.shape[1]), v.dtype)], axis=1)


def _make_mid(Din, H):
    def body(h_ref, p_ref, dis_ref, w0_ref, w1_ref, acc_ref, g1_ref):
        d = dis_ref[...][:, 0:1]
        s = (p_ref[0] + p_ref[1])[:, :Din]
        tx1 = -(d * s)
        g1_ref[...] = _padw(d * tx1)
        acc_ref[...] = (
            jnp.dot(h_ref[...], w0_ref[...], preferred_element_type=jnp.float32)
            + jnp.dot(tx1, w1_ref[...], preferred_element_type=jnp.float32)
        )

    return pl.pallas_call(
        body,
        grid=(N // _R,),
        in_specs=[
            pl.BlockSpec((_R, Din), lambda i: (i, 0)),
            pl.BlockSpec((2, _R, DP), lambda i: (0, i, 0)),
            pl.BlockSpec((_R, DW), lambda i: (i, 0)),
            pl.BlockSpec((Din, H), lambda i: (0, 0)),
            pl.BlockSpec((Din, H), lambda i: (0, 0)),
        ],
        out_specs=[
            pl.BlockSpec((_R, H), lambda i: (i, 0)),
            pl.BlockSpec((_R, DP), lambda i: (i, 0)),
        ],
        out_shape=[
            jax.ShapeDtypeStruct((N, H), jnp.float32),
            jax.ShapeDtypeStruct((N, DP), jnp.float32),
        ],
    )


def _make_post(Din, H, relu):
    def body(h_ref, p_ref, acc_ref, dis_ref, w2_ref, b_ref, out_ref,
             gn_ref=None):
        d = dis_ref[...][:, 0:1]
        tx2 = -2.0 * d * (p_ref[0] + p_ref[1])[:, :Din] - h_ref[...]
        o = (acc_ref[...]
             + jnp.dot(tx2, w2_ref[...], preferred_element_type=jnp.float32)
             + b_ref[...])
        if relu:
            hp = jnp.maximum(o, 0.0)
            out_ref[...] = hp
            gn_ref[...] = _padw(d * hp)
        else:
            out_ref[...] = o

    out_specs = [pl.BlockSpec((_R, H), lambda i: (i, 0)),
                 pl.BlockSpec((_R, DP), lambda i: (i, 0))]
    out_shape = [jax.ShapeDtypeStruct((N, H), jnp.float32),
                 jax.ShapeDtypeStruct((N, DP), jnp.float32)]
    return pl.pallas_call(
        body,
        grid=(N // _R,),
        in_specs=[
            pl.BlockSpec((_R, Din), lambda i: (i, 0)),
            pl.BlockSpec((2, _R, DP), lambda i: (0, i, 0)),
            pl.BlockSpec((_R, H), lambda i: (i, 0)),
            pl.BlockSpec((_R, DW), lambda i: (i, 0)),
            pl.BlockSpec((Din, H), lambda i: (0, 0)),
            pl.BlockSpec((1, H), lambda i: (0, 0)),
        ],
        out_specs=out_specs if relu else out_specs[0],
        out_shape=out_shape if relu else out_shape[0],
    )


_dis_g0_call = _make_dis_g0(128)
_mid_calls = [_make_mid(128, 64), _make_mid(64, 128), _make_mid(128, 128)]
_post_calls = [_make_post(128, 64, True), _make_post(64, 128, True),
               _make_post(128, 128, False)]


def kernel(x, edge_index, batch, W1, b1, W2, b2, W3, b3):
    pad = E_PAD - E
    row = jnp.concatenate([edge_index[0], jnp.zeros((pad,), jnp.int32)])
    col = jnp.concatenate([edge_index[1], jnp.zeros((pad,), jnp.int32)])
    rowg, rowd, cols = _prep_call(row.reshape(EP_ROWS, CH),
                                  col.reshape(EP_ROWS, CH))

    z128 = jnp.zeros((CH, DP), jnp.float32)

    ones_tbl = jnp.ones((N, DP), jnp.float32)
    degp = _make_prop_kernel(DP)(ones_tbl, rowg, rowd, z128)
    dis, g = _dis_g0_call(degp, x)

    ws = [(W1, b1), (W2, b2), (W3, b3)]
    h = x
    out = None
    for li, (W, b) in enumerate(ws):
        p1 = _make_prop_kernel(DP)(g, rowg, cols, z128)
        acc, g = _mid_calls[li](h, p1, dis, W[0], W[1])
        p2 = _make_prop_kernel(DP)(g, rowg, cols, z128)
        res = _post_calls[li](h, p2, acc, dis, W[2], b.reshape(1, -1))
        if li < 2:
            h, g = res
        else:
            out = res
    return out


# speedup vs baseline: 13.9746x; 1.0453x over previous
import functools

import jax
import jax.numpy as jnp
from jax import lax
from jax.experimental import pallas as pl
from jax.experimental.pallas import tpu as pltpu
from jax.experimental.pallas import tpu_sc as plsc

N = 10000
E = 320000
NC = 2
NS = 16
NW = NC * NS
CH = 128
CHUNKS_W = 80
CG = 40
E_PAD = NW * CH * CHUNKS_W
DW = 16
EP_ROWS = E_PAD // CH
N_ACC = 10240
ROWS_T = N_ACC // NS
HOPS = ROWS_T // CH

@functools.cache
def _get_mesh():
    return plsc.VectorSubcoreMesh(core_axis_name="c", subcore_axis_name="s",
                                  num_cores=NC, num_subcores=NS)


def _zero_acc(z_hbm, stage, acc, sid):
    pltpu.sync_copy(z_hbm, stage)
    for j in range(HOPS):
        pltpu.sync_copy(stage, acc.at[pl.ds(sid * ROWS_T + j * CH, CH)])


def _drain_acc(acc, stage, out_hbm, cid, sid):
    for j in range(HOPS):
        sl = pl.ds(sid * ROWS_T + j * CH, CH)
        pltpu.sync_copy(acc.at[sl], stage)
        pltpu.sync_copy(stage, out_hbm.at[cid, sl])


@functools.cache
def _make_deg_stream_kernel():
    @functools.partial(
        pl.kernel,
        out_type=jax.ShapeDtypeStruct((NC, N_ACC, DP), jnp.float32),
        mesh=_get_mesh(),
        scratch_types=[
            pltpu.VMEM((CHUNKS_W, CH), jnp.int32),
            pltpu.VMEM((CH, DP), jnp.float32),
            pltpu.VMEM((CH, DP), jnp.float32),
            pltpu.VMEM_SHARED((N_ACC, DP), jnp.float32),
        ],
    )
    def deg_kernel(rowd_hbm, one_hbm, z_hbm, out_hbm, idxs, upd, zb, acc):
        cid = lax.axis_index("c")
        sid = lax.axis_index("s")
        wid = sid * NC + cid
        pltpu.sync_copy(z_hbm, zb)
        for j in range(HOPS):
            pltpu.sync_copy(zb, acc.at[pl.ds(sid * ROWS_T + j * CH, CH)])
        pltpu.sync_copy(one_hbm, upd)
        pltpu.sync_copy(rowd_hbm.at[pl.ds(wid * CHUNKS_W, CHUNKS_W)], idxs)
        plsc.subcore_barrier()

        def body(c, _):
            pltpu.sync_copy(upd, acc.at[idxs.at[c]], add=True)
            return 0

        lax.fori_loop(0, CHUNKS_W, body, 0)
        plsc.subcore_barrier()
        _drain_acc(acc, upd, out_hbm, cid, sid)

    return deg_kernel


@functools.cache
def _make_prop_kernel(D):
    @functools.partial(
        pl.kernel,
        out_type=jax.ShapeDtypeStruct((NC, N_ACC, D), jnp.float32),
        mesh=_get_mesh(),
        scratch_types=[
            pltpu.VMEM((CG, CH), jnp.int32),
            pltpu.VMEM((CG, CH), jnp.int32),
            pltpu.VMEM((CH, D), jnp.float32),
            pltpu.VMEM((CH, D), jnp.float32),
            pltpu.VMEM_SHARED((N_ACC, D), jnp.float32),
            pltpu.SemaphoreType.DMA,
            pltpu.SemaphoreType.DMA,
        ],
    )
    def prop_kernel(g_hbm, rowg_hbm, cols_hbm, z_hbm, out_hbm,
                    idxr, idxc, buf0, buf1, acc, sem0, sem1):
        cid = lax.axis_index("c")
        sid = lax.axis_index("s")
        wid = sid * NC + cid
        base = wid * CHUNKS_W
        _zero_acc(z_hbm, buf0, acc, sid)
        plsc.subcore_barrier()

        def step(c, buf, sem, start_next):
            pltpu.make_async_copy(g_hbm.at[idxr.at[c]], buf, sem).wait()
            pltpu.sync_copy(buf, acc.at[idxc.at[c]], add=True)
            if start_next:
                pltpu.async_copy(g_hbm.at[idxr.at[c + 2]], buf, sem)

        def body(k, _):
            c = 2 * k
            step(c, buf0, sem0, True)
            step(c + 1, buf1, sem1, True)
            return 0

        for gi in range(CHUNKS_W // CG):
            pltpu.sync_copy(rowg_hbm.at[pl.ds(base + gi * CG, CG)], idxr)
            pltpu.sync_copy(cols_hbm.at[pl.ds(base + gi * CG, CG)], idxc)
            pltpu.async_copy(g_hbm.at[idxr.at[0]], buf0, sem0)
            pltpu.async_copy(g_hbm.at[idxr.at[1]], buf1, sem1)
            lax.fori_loop(0, CG // 2 - 1, body, 0)
            step(CG - 2, buf0, sem0, False)
            step(CG - 1, buf1, sem1, False)
        plsc.subcore_barrier()
        _drain_acc(acc, buf0, out_hbm, cid, sid)

    return prop_kernel


_RB = 256
_R = 400
_RD = 640


def _prep_body(row_ref, col_ref, rowg_ref, rowd_ref, cols_ref):
    i = pl.program_id(0)
    r = row_ref[...]
    c = col_ref[...]
    ri = lax.broadcasted_iota(jnp.int32, (_RB, CH), 0)
    ci = lax.broadcasted_iota(jnp.int32, (_RB, CH), 1)
    pos = (i * _RB + ri) * CH + ci
    self_m = r == c
    trash = N + (pos & 127)
    spread = pos & 8191
    rowg_ref[...] = jnp.where(self_m, spread, r)
    rowd_ref[...] = jnp.where(self_m, trash, r)
    cols_ref[...] = jnp.where(self_m, trash, c)


_prep_call = pl.pallas_call(
    _prep_body,
    grid=(EP_ROWS // _RB,),
    in_specs=[pl.BlockSpec((_RB, CH), lambda i: (i, 0))] * 2,
    out_specs=[pl.BlockSpec((_RB, CH), lambda i: (i, 0))] * 3,
    out_shape=[jax.ShapeDtypeStruct((EP_ROWS, CH), jnp.int32)] * 3,
)


def _dis_g0_body(degp_ref, x_ref, dis_ref, g_ref):
    deg = (degp_ref[0] + degp_ref[1])[:, 0:1]
    ok = deg > 0.0
    dis = jnp.where(ok, lax.rsqrt(jnp.where(ok, deg, 1.0)), 0.0)
    dis_ref[...] = jnp.broadcast_to(dis, (dis.shape[0], DW))
    g_ref[...] = x_ref[...] * dis


def _make_dis_g0(Din):
    return pl.pallas_call(
        _dis_g0_body,
        grid=(N_ACC // _RD,),
        in_specs=[
            pl.BlockSpec((2, _RD, DP), lambda i: (0, i, 0)),
            pl.BlockSpec((_RD, Din), lambda i: (i, 0)),
        ],
        out_specs=[
            pl.BlockSpec((_RD, DW), lambda i: (i, 0)),
            pl.BlockSpec((_RD, Din), lambda i: (i, 0)),
        ],
        out_shape=[
            jax.ShapeDtypeStruct((N_ACC, DW), jnp.float32),
            jax.ShapeDtypeStruct((N, Din), jnp.float32),
        ],
    )


DP = 128


def _padw(v):
    if v.shape[1] == DP:
        return v
    return jnp.concatenate(
        [v, jnp.zeros((v.shape[0], DP - v.shape[1]), v.dtype)], axis=1)


def _make_mid(Din, H):
    def body(h_ref, p_ref, dis_ref, w0_ref, w1_ref, acc_ref, g1_ref):
        d = dis_ref[...][:, 0:1]
        s = (p_ref[0] + p_ref[1])[:, :Din]
        tx1 = -(d * s)
        g1_ref[...] = _padw(d * tx1)
        acc_ref[...] = (
            jnp.dot(h_ref[...], w0_ref[...], preferred_element_type=jnp.float32)
            + jnp.dot(tx1, w1_ref[...], preferred_element_type=jnp.float32)
        )

    return pl.pallas_call(
        body,
        grid=(N // _R,),
        in_specs=[
            pl.BlockSpec((_R, Din), lambda i: (i, 0)),
            pl.BlockSpec((2, _R, DP), lambda i: (0, i, 0)),
            pl.BlockSpec((_R, DW), lambda i: (i, 0)),
            pl.BlockSpec((Din, H), lambda i: (0, 0)),
            pl.BlockSpec((Din, H), lambda i: (0, 0)),
        ],
        out_specs=[
            pl.BlockSpec((_R, H), lambda i: (i, 0)),
            pl.BlockSpec((_R, DP), lambda i: (i, 0)),
        ],
        out_shape=[
            jax.ShapeDtypeStruct((N, H), jnp.float32),
            jax.ShapeDtypeStruct((N, DP), jnp.float32),
        ],
    )


def _make_post(Din, H, relu):
    def body(h_ref, p_ref, acc_ref, dis_ref, w2_ref, b_ref, out_ref,
             gn_ref=None):
        d = dis_ref[...][:, 0:1]
        tx2 = -2.0 * d * (p_ref[0] + p_ref[1])[:, :Din] - h_ref[...]
        o = (acc_ref[...]
             + jnp.dot(tx2, w2_ref[...], preferred_element_type=jnp.float32)
             + b_ref[...])
        if relu:
            hp = jnp.maximum(o, 0.0)
            out_ref[...] = hp
            gn_ref[...] = _padw(d * hp)
        else:
            out_ref[...] = o

    out_specs = [pl.BlockSpec((_R, H), lambda i: (i, 0)),
                 pl.BlockSpec((_R, DP), lambda i: (i, 0))]
    out_shape = [jax.ShapeDtypeStruct((N, H), jnp.float32),
                 jax.ShapeDtypeStruct((N, DP), jnp.float32)]
    return pl.pallas_call(
        body,
        grid=(N // _R,),
        in_specs=[
            pl.BlockSpec((_R, Din), lambda i: (i, 0)),
            pl.BlockSpec((2, _R, DP), lambda i: (0, i, 0)),
            pl.BlockSpec((_R, H), lambda i: (i, 0)),
            pl.BlockSpec((_R, DW), lambda i: (i, 0)),
            pl.BlockSpec((Din, H), lambda i: (0, 0)),
            pl.BlockSpec((1, H), lambda i: (0, 0)),
        ],
        out_specs=out_specs if relu else out_specs[0],
        out_shape=out_shape if relu else out_shape[0],
    )


_dis_g0_call = _make_dis_g0(128)
_mid_calls = [_make_mid(128, 64), _make_mid(64, 128), _make_mid(128, 128)]
_post_calls = [_make_post(128, 64, True), _make_post(64, 128, True),
               _make_post(128, 128, False)]


def kernel(x, edge_index, batch, W1, b1, W2, b2, W3, b3):
    pad = E_PAD - E
    row = jnp.concatenate([edge_index[0], jnp.zeros((pad,), jnp.int32)])
    col = jnp.concatenate([edge_index[1], jnp.zeros((pad,), jnp.int32)])
    rowg, rowd, cols = _prep_call(row.reshape(EP_ROWS, CH),
                                  col.reshape(EP_ROWS, CH))

    z128 = jnp.zeros((CH, DP), jnp.float32)

    ones_upd = jnp.ones((CH, DP), jnp.float32)
    degp = _make_deg_stream_kernel()(rowd, ones_upd, z128)
    dis, g = _dis_g0_call(degp, x)

    ws = [(W1, b1), (W2, b2), (W3, b3)]
    h = x
    out = None
    for li, (W, b) in enumerate(ws):
        p1 = _make_prop_kernel(DP)(g, rowg, cols, z128)
        acc, g = _mid_calls[li](h, p1, dis, W[0], W[1])
        p2 = _make_prop_kernel(DP)(g, rowg, cols, z128)
        res = _post_calls[li](h, p2, acc, dis, W[2], b.reshape(1, -1))
        if li < 2:
            h, g = res
        else:
            out = res
    return out
